# baseline (speedup 1.0000x reference)
import functools

import numpy as np
import jax
import jax.numpy as jnp
from jax import lax
from jax.experimental import pallas as pl
from jax.experimental.pallas import tpu as pltpu
from jax.experimental.pallas import tpu_sc as plsc

_B = 16384
_NF = 26
_V = 100000
_D = 32
_NI = _NF + 1
_BNF = _B * _NF
_NW = 32
_RPW = _BNF // _NW
_CH = 128
_NCH = _RPW // _CH
_R = 512

_TRI = np.array([i * _NI + j for i in range(_NI) for j in range(i)], dtype=np.int32)


def _sc_gather(tables_flat, idx3):
    mesh = plsc.VectorSubcoreMesh(core_axis_name="c", subcore_axis_name="s")

    @functools.partial(
        pl.kernel,
        mesh=mesh,
        out_type=jax.ShapeDtypeStruct((_BNF, _D), jnp.bfloat16),
        compiler_params=pltpu.CompilerParams(use_tc_tiling_on_sc=False),
        scratch_types=[
            pltpu.VMEM((_NCH, _CH), jnp.int32),
            pltpu.VMEM((_CH, _D), jnp.bfloat16),
            pltpu.VMEM((_CH, _D), jnp.bfloat16),
            pltpu.SemaphoreType.DMA,
            pltpu.SemaphoreType.DMA,
        ],
    )
    def k(tab_hbm, idx_hbm, out_hbm, idx_v, buf0, buf1, sem0, sem1):
        wid = lax.axis_index("s") * 2 + lax.axis_index("c")
        base = wid * _RPW
        pltpu.sync_copy(idx_hbm.at[wid], idx_v)
        pltpu.async_copy(tab_hbm.at[idx_v.at[0]], buf0, sem0)
        pltpu.async_copy(tab_hbm.at[idx_v.at[1]], buf1, sem1)

        def body(kk, _):
            c0 = 2 * kk

            def step(buf, sem, c):
                pltpu.make_async_copy(tab_hbm.at[idx_v.at[c]], buf, sem).wait()
                pltpu.sync_copy(buf, out_hbm.at[pl.ds(base + c * _CH, _CH)])

                @pl.when(c + 2 < _NCH)
                def _():
                    pltpu.async_copy(tab_hbm.at[idx_v.at[c + 2]], buf, sem)

            step(buf0, sem0, c0)
            step(buf1, sem1, c0 + 1)
            return ()

        lax.fori_loop(0, _NCH // 2, body, (), unroll=False)

    return k(tables_flat, idx3)


def _tc_body(num_ref, gat_ref, wb0, bb0, wb1, bb1, wb2, bb2,
             w0a, w0i, bt0, wt1, bt1, wt2, bt2, wt3, bt3, wt4, bt4,
             out_ref):
    f32 = jnp.float32
    bf16 = jnp.bfloat16
    x = num_ref[...]
    x = jnp.maximum(jnp.dot(x, wb0[...], preferred_element_type=f32) + bb0[...], 0.0)
    x = jnp.maximum(jnp.dot(x.astype(bf16), wb1[...], preferred_element_type=f32) + bb1[...], 0.0)
    bot = jnp.maximum(jnp.dot(x.astype(bf16), wb2[...], preferred_element_type=f32) + bb2[...], 0.0)
    botb = bot.astype(bf16)
    C3 = jnp.concatenate([botb, gat_ref[...]], axis=1).reshape(_R, _NI, _D)
    inter = lax.dot_general(C3, C3, (((2,), (2,)), ((0,), (0,))),
                            preferred_element_type=f32)
    interf = inter.reshape(_R, _NI * _NI).astype(bf16)
    y = jnp.dot(botb, w0a[...], preferred_element_type=f32)
    y = y + jnp.dot(interf, w0i[...], preferred_element_type=f32)
    y = jnp.maximum(y + bt0[...], 0.0)
    y = jnp.maximum(jnp.dot(y.astype(bf16), wt1[...], preferred_element_type=f32) + bt1[...], 0.0)
    y = jnp.maximum(jnp.dot(y.astype(bf16), wt2[...], preferred_element_type=f32) + bt2[...], 0.0)
    y = jnp.maximum(jnp.dot(y.astype(bf16), wt3[...], preferred_element_type=f32) + bt3[...], 0.0)
    out_ref[...] = jnp.dot(y.astype(bf16), wt4[...], preferred_element_type=f32) + bt4[...]


def kernel(numerical_input, categorical_inputs, emb_tables,
           W_bot_0, b_bot_0, W_bot_1, b_bot_1, W_bot_2, b_bot_2,
           W_top_0, b_top_0, W_top_1, b_top_1, W_top_2, b_top_2,
           W_top_3, b_top_3, W_top_4, b_top_4):
    offs = (jnp.arange(_NF, dtype=jnp.int32) * _V)[None, :]
    idx3 = (categorical_inputs + offs).reshape(_NW, _NCH, _CH)
    bf16 = jnp.bfloat16
    tab_bf = emb_tables.astype(bf16).reshape(_NF * _V, _D)
    gathered = _sc_gather(tab_bf, idx3)
    gat2 = gathered.reshape(_B, _NF * _D)

    w0a = W_top_0[:_D].astype(bf16)
    w0i = jnp.zeros((_NI * _NI, W_top_0.shape[1]), bf16)
    w0i = w0i.at[_TRI].set(W_top_0[_D:_D + _TRI.shape[0]].astype(bf16))

    row = lambda b: b.reshape(1, -1)
    grid = _B // _R
    full = lambda a: pl.BlockSpec(a.shape, lambda i: (0,) * a.ndim)
    out = pl.pallas_call(
        _tc_body,
        grid=(grid,),
        in_specs=[
            pl.BlockSpec((_R, numerical_input.shape[1]), lambda i: (i, 0)),
            pl.BlockSpec((_R, _NF * _D), lambda i: (i, 0)),
            full(W_bot_0), full(row(b_bot_0)), full(W_bot_1), full(row(b_bot_1)),
            full(W_bot_2), full(row(b_bot_2)),
            full(w0a), full(w0i), full(row(b_top_0)),
            full(W_top_1), full(row(b_top_1)), full(W_top_2), full(row(b_top_2)),
            full(W_top_3), full(row(b_top_3)), full(W_top_4), full(row(b_top_4)),
        ],
        out_specs=pl.BlockSpec((_R, 1), lambda i: (i, 0)),
        out_shape=jax.ShapeDtypeStruct((_B, 1), jnp.float32),
    )(numerical_input.astype(bf16), gat2,
      W_bot_0.astype(bf16), row(b_bot_0), W_bot_1.astype(bf16), row(b_bot_1),
      W_bot_2.astype(bf16), row(b_bot_2),
      w0a, w0i, row(b_top_0), W_top_1.astype(bf16), row(b_top_1),
      W_top_2.astype(bf16), row(b_top_2),
      W_top_3.astype(bf16), row(b_top_3), W_top_4.astype(bf16), row(b_top_4))
    return out


# speedup vs baseline: 1.5309x; 1.5309x over previous
import functools

import numpy as np
import jax
import jax.numpy as jnp
from jax import lax
from jax.experimental import pallas as pl
from jax.experimental.pallas import tpu as pltpu
from jax.experimental.pallas import tpu_sc as plsc

_B = 16384
_NF = 26
_V = 100000
_D = 32
_NI = _NF + 1
_BNF = _B * _NF
_R = 512

_VC = 768
_NCHV = 132
_VLAST = 128
_TAILB = 99968
_NB = 160
_G = 256
_NG = _B // _G

_TRI = np.array([i * _NI + j for i in range(_NI) for j in range(i)], dtype=np.int32)


def _sweep_gather(embT3, catT, tail):
    mesh = plsc.VectorSubcoreMesh(core_axis_name="c", subcore_axis_name="s")

    @functools.partial(
        pl.kernel,
        mesh=mesh,
        out_type=(jax.ShapeDtypeStruct((_BNF // 4, 128), jnp.float32),
                  jax.ShapeDtypeStruct((_NF * _NG, _G), jnp.int32)),
        compiler_params=pltpu.CompilerParams(needs_layout_passes=False),
        scratch_types=[
            pltpu.VMEM((128, 128), jnp.int32),
            pltpu.VMEM((16 * _NB,), jnp.int32),
            pltpu.VMEM((16 * _NB,), jnp.int32),
            pltpu.VMEM((_NB,), jnp.int32),
            pltpu.VMEM((_B + 16,), jnp.int32),
            pltpu.VMEM((_NG, _G), jnp.int32),
            pltpu.VMEM((2, _D, _VC), jnp.float32),
            pltpu.VMEM((2 * _G // 4, 128), jnp.float32),
            pltpu.SemaphoreType.DMA,
            pltpu.SemaphoreType.DMA,
        ],
    )
    def k(emb_hbm, cat_hbm, tail_hbm, srt_hbm, bidx_hbm, idx_v, hist, cur,
          goff, sortv, bidx, chunks, stag, csem, fsem):
        wid = lax.axis_index("s") * 2 + lax.axis_index("c")
        f = wid

        @pl.when(f < _NF)
        def _work():
            pltpu.sync_copy(cat_hbm.at[f], idx_v)
            lanes = lax.iota(jnp.int32, 16)
            lane_off = lanes * _NB
            zeros16 = jnp.zeros((16,), jnp.int32)
            ones16 = jnp.ones((16,), jnp.int32)

            def zh(i, _):
                hist[pl.ds(i * 16, 16)] = zeros16
                return ()
            lax.fori_loop(0, 16 * _NB // 16, zh, (), unroll=False)

            def buckets_of(i):
                v = idx_v[i >> 3, pl.ds((i & 7) * 16, 16)]
                c = jnp.minimum(((v >> 8) * 683) >> 11, 130)
                c = jnp.where(v >= _TAILB, 131, c)
                base = jnp.where(c == 131, _TAILB, c * _VC)
                voff = v - base
                return c, voff

            def h1(i, _):
                c, _voff = buckets_of(i)
                slot = lane_off + c
                cnt = plsc.load_gather(hist, [slot])
                plsc.store_scatter(hist, [slot], cnt + ones16)
                return ()
            lax.fori_loop(0, _B // 16, h1, (), unroll=False)

            def red(g, carry):
                acc = zeros16

                def rl(l, a):
                    return a + hist[pl.ds(l * _NB + g * 16, 16)]
                acc = lax.fori_loop(0, 16, rl, acc, unroll=False)
                cs = plsc.cumsum(acc)
                excl = cs - acc + jnp.full((16,), carry, jnp.int32)
                goff[pl.ds(g * 16, 16)] = excl
                def wl(l, run):
                    cur[pl.ds(l * _NB + g * 16, 16)] = run
                    return run + hist[pl.ds(l * _NB + g * 16, 16)]
                lax.fori_loop(0, 16, wl, excl, unroll=False)
                return carry + cs[15]
            lax.fori_loop(0, _NB // 16, red, jnp.int32(0), unroll=False)

            bbase = f * _B

            def h2(i, _):
                c, voff = buckets_of(i)
                bglob = jnp.full((16,), bbase + i * 16, jnp.int32) + lanes
                packed = (bglob << 10) | voff
                slot = lane_off + c
                pos = plsc.load_gather(cur, [slot])
                plsc.store_scatter(sortv, [pos], packed)
                plsc.store_scatter(bidx, [pos >> 8, pos & (_G - 1)], bglob)
                plsc.store_scatter(cur, [slot], pos + 1)
                return ()
            lax.fori_loop(0, _B // 16, h2, (), unroll=False)

            pltpu.async_copy(emb_hbm.at[f, :, pl.ds(0, _VC)], chunks.at[0], csem)

            def start_chunk(c, buf):
                @pl.when(c <= 129)
                def _full():
                    pltpu.async_copy(
                        emb_hbm.at[f, :, pl.ds(pl.multiple_of(c * _VC, _VC), _VC)],
                        chunks.at[buf], csem)

                @pl.when(c == 130)
                def _short():
                    pltpu.async_copy(
                        emb_hbm.at[f, :, pl.ds(130 * _VC, _VLAST)],
                        chunks.at[buf, :, pl.ds(0, _VLAST)], csem)

                @pl.when(c == 131)
                def _tail():
                    pltpu.async_copy(
                        tail_hbm.at[f],
                        chunks.at[buf, :, pl.ds(0, 128)], csem)

            def wait_chunk(c, buf):
                @pl.when(c <= 129)
                def _full():
                    pltpu.make_async_copy(
                        emb_hbm.at[f, :, pl.ds(0, _VC)],
                        chunks.at[buf], csem).wait()

                @pl.when(c == 130)
                def _short():
                    pltpu.make_async_copy(
                        emb_hbm.at[f, :, pl.ds(130 * _VC, _VLAST)],
                        chunks.at[buf, :, pl.ds(0, _VLAST)], csem).wait()

                @pl.when(c == 131)
                def _tail():
                    pltpu.make_async_copy(
                        tail_hbm.at[f],
                        chunks.at[buf, :, pl.ds(0, 128)], csem).wait()

            def sweep(c, _):
                par = c & 1

                @pl.when(c + 1 < _NCHV)
                def _pref():
                    start_chunk(c + 1, 1 - par)
                wait_chunk(c, par)

                gv = goff[pl.ds(c, 16)]
                lo = gv[0]
                hi = gv[1]
                par16 = jnp.full((16,), par, jnp.int32)

                def item(kk, _):
                    packed = sortv[pl.ds(kk, 16)][0]
                    voff16 = jnp.full((16,), packed & 1023, jnp.int32)
                    g0 = plsc.load_gather(chunks, [par16, lanes, voff16])
                    g1 = plsc.load_gather(
                        chunks, [par16, lanes + 16, voff16])
                    kw = kk & (2 * _G - 1)
                    srow = jnp.full((16,), kw >> 2, jnp.int32)
                    l0 = jnp.full((16,), (kw & 3) * _D, jnp.int32) + lanes
                    plsc.store_scatter(stag, [srow, l0], g0)
                    plsc.store_scatter(stag, [srow, l0 + 16], g1)

                    @pl.when((kk & (_G - 1)) == (_G - 1))
                    def _flush():
                        j = kk >> 8
                        jpar = j & 1
                        dsto = pl.multiple_of((f * _B + j * _G) // 4, 64)
                        pltpu.async_copy(
                            stag.at[pl.ds(jpar * (_G // 4), _G // 4)],
                            srt_hbm.at[pl.ds(dsto, _G // 4)],
                            fsem)

                        @pl.when(j >= 1)
                        def _drain():
                            pltpu.make_async_copy(
                                stag.at[pl.ds(0, _G // 4)],
                                srt_hbm.at[pl.ds(0, _G // 4)], fsem).wait()
                    return ()
                lax.fori_loop(lo, hi, item, (), unroll=False)
                return ()
            lax.fori_loop(0, _NCHV, sweep, (), unroll=False)
            pltpu.make_async_copy(
                stag.at[pl.ds(0, _G // 4)],
                srt_hbm.at[pl.ds(0, _G // 4)], fsem).wait()
            pltpu.sync_copy(
                bidx, bidx_hbm.at[pl.ds(pl.multiple_of(f * _NG, _NG), _NG)])

    return k(embT3, catT, tail)


def _perm_scatter(srt, bidxh):
    mesh = plsc.VectorSubcoreMesh(core_axis_name="c", subcore_axis_name="s")
    _U = 128
    upw = _BNF // _U // 32

    @functools.partial(
        pl.kernel,
        mesh=mesh,
        out_type=jax.ShapeDtypeStruct((_BNF, _D), jnp.float32),
        compiler_params=pltpu.CompilerParams(
            use_tc_tiling_on_sc=False, needs_layout_passes=False),
        scratch_types=[
            pltpu.VMEM((upw, _U), jnp.int32),
            pltpu.VMEM((2, _U, _D), jnp.float32),
            pltpu.SemaphoreType.DMA,
            pltpu.SemaphoreType.DMA,
        ],
    )
    def k(srt_hbm, bidx_hbm, out_hbm, bidx_v, bufs, lsem, ssem):
        wid = lax.axis_index("s") * 2 + lax.axis_index("c")
        g0 = wid * upw
        pltpu.sync_copy(bidx_hbm.at[pl.ds(g0, upw)], bidx_v)
        pltpu.async_copy(srt_hbm.at[pl.ds(g0 * _U, _U)], bufs.at[0], lsem)

        def body(g, _):
            par = g & 1

            @pl.when(g >= 1)
            def _drain():
                pltpu.make_async_copy(
                    bufs.at[0], out_hbm.at[bidx_v.at[0]], ssem).wait()

            @pl.when(g + 1 < upw)
            def _pref():
                pltpu.async_copy(
                    srt_hbm.at[pl.ds((g0 + g + 1) * _U, _U)],
                    bufs.at[1 - par], lsem)
            pltpu.make_async_copy(
                srt_hbm.at[pl.ds(0, _U)], bufs.at[par], lsem).wait()
            pltpu.async_copy(bufs.at[par], out_hbm.at[bidx_v.at[g]], ssem)
            return ()
        lax.fori_loop(0, upw, body, (), unroll=False)
        pltpu.make_async_copy(
            bufs.at[0], out_hbm.at[bidx_v.at[0]], ssem).wait()

    return k(srt, bidxh)


def _tc_body(num_ref, gat_ref, wb0, bb0, wb1, bb1, wb2, bb2,
             w0a, w0i, bt0, wt1, bt1, wt2, bt2, wt3, bt3, wt4, bt4,
             out_ref):
    f32 = jnp.float32
    bf16 = jnp.bfloat16
    x = num_ref[...]
    x = jnp.maximum(jnp.dot(x, wb0[...], preferred_element_type=f32) + bb0[...], 0.0)
    x = jnp.maximum(jnp.dot(x.astype(bf16), wb1[...], preferred_element_type=f32) + bb1[...], 0.0)
    bot = jnp.maximum(jnp.dot(x.astype(bf16), wb2[...], preferred_element_type=f32) + bb2[...], 0.0)
    botb = bot.astype(bf16)
    gat = gat_ref[...].astype(bf16)
    A = jnp.concatenate([botb[None], gat], axis=0)
    inter = lax.dot_general(A, A, (((2,), (2,)), ((1,), (1,))),
                            preferred_element_type=f32)
    interf = inter.reshape(_R, _NI * _NI).astype(bf16)
    y = jnp.dot(botb, w0a[...], preferred_element_type=f32)
    y = y + jnp.dot(interf, w0i[...], preferred_element_type=f32)
    y = jnp.maximum(y + bt0[...], 0.0)
    y = jnp.maximum(jnp.dot(y.astype(bf16), wt1[...], preferred_element_type=f32) + bt1[...], 0.0)
    y = jnp.maximum(jnp.dot(y.astype(bf16), wt2[...], preferred_element_type=f32) + bt2[...], 0.0)
    y = jnp.maximum(jnp.dot(y.astype(bf16), wt3[...], preferred_element_type=f32) + bt3[...], 0.0)
    out_ref[...] = jnp.dot(y.astype(bf16), wt4[...], preferred_element_type=f32) + bt4[...]


def kernel(numerical_input, categorical_inputs, emb_tables,
           W_bot_0, b_bot_0, W_bot_1, b_bot_1, W_bot_2, b_bot_2,
           W_top_0, b_top_0, W_top_1, b_top_1, W_top_2, b_top_2,
           W_top_3, b_top_3, W_top_4, b_top_4):
    bf16 = jnp.bfloat16
    embT3 = jnp.transpose(emb_tables, (0, 2, 1))
    catT = jnp.transpose(categorical_inputs).reshape(_NF, 128, _B // 128)
    tail = jnp.pad(embT3[:, :, _TAILB:], ((0, 0), (0, 0), (0, 96)))
    srt4, bidxh = _sweep_gather(embT3, catT, tail)
    gathered = _perm_scatter(srt4.reshape(_BNF, _D),
                             bidxh.reshape(_BNF // 128, 128))
    gat3 = gathered.reshape(_NF, _B, _D)

    w0a = W_top_0[:_D].astype(bf16)
    w0i = jnp.zeros((_NI * _NI, W_top_0.shape[1]), bf16)
    w0i = w0i.at[_TRI].set(W_top_0[_D:_D + _TRI.shape[0]].astype(bf16))

    row = lambda b: b.reshape(1, -1)
    grid = _B // _R
    full = lambda a: pl.BlockSpec(a.shape, lambda i: (0,) * a.ndim)
    out = pl.pallas_call(
        _tc_body,
        grid=(grid,),
        in_specs=[
            pl.BlockSpec((_R, numerical_input.shape[1]), lambda i: (i, 0)),
            pl.BlockSpec((_NF, _R, _D), lambda i: (0, i, 0)),
            full(W_bot_0), full(row(b_bot_0)), full(W_bot_1), full(row(b_bot_1)),
            full(W_bot_2), full(row(b_bot_2)),
            full(w0a), full(w0i), full(row(b_top_0)),
            full(W_top_1), full(row(b_top_1)), full(W_top_2), full(row(b_top_2)),
            full(W_top_3), full(row(b_top_3)), full(W_top_4), full(row(b_top_4)),
        ],
        out_specs=pl.BlockSpec((_R, 1), lambda i: (i, 0)),
        out_shape=jax.ShapeDtypeStruct((_B, 1), jnp.float32),
    )(numerical_input.astype(bf16), gat3,
      W_bot_0.astype(bf16), row(b_bot_0), W_bot_1.astype(bf16), row(b_bot_1),
      W_bot_2.astype(bf16), row(b_bot_2),
      w0a, w0i, row(b_top_0), W_top_1.astype(bf16), row(b_top_1),
      W_top_2.astype(bf16), row(b_top_2),
      W_top_3.astype(bf16), row(b_top_3), W_top_4.astype(bf16), row(b_top_4))
    return out


# speedup vs baseline: 1.8035x; 1.1780x over previous
import functools

import numpy as np
import jax
import jax.numpy as jnp
from jax import lax
from jax.experimental import pallas as pl
from jax.experimental.pallas import tpu as pltpu
from jax.experimental.pallas import tpu_sc as plsc

_B = 16384
_NF = 26
_V = 100000
_D = 32
_NI = _NF + 1
_BNF = _B * _NF
_R = 512

_VC = 768
_NCHV = 132
_VLAST = 128
_TAILB = 99968
_NB = 160
_G = 256
_NG = _B // _G

_TRI = np.array([i * _NI + j for i in range(_NI) for j in range(i)], dtype=np.int32)


def _sweep_gather(embT3, catT, tail):
    mesh = plsc.VectorSubcoreMesh(core_axis_name="c", subcore_axis_name="s")

    @functools.partial(
        pl.kernel,
        mesh=mesh,
        out_type=(jax.ShapeDtypeStruct((_BNF // 4, 128), jnp.float32),
                  jax.ShapeDtypeStruct((_NF * _NG, _G), jnp.int32)),
        compiler_params=pltpu.CompilerParams(needs_layout_passes=False),
        scratch_types=[
            pltpu.VMEM((128, 128), jnp.int32),
            pltpu.VMEM((16 * _NB,), jnp.int32),
            pltpu.VMEM((16 * _NB,), jnp.int32),
            pltpu.VMEM((_NB,), jnp.int32),
            pltpu.VMEM((_B + 16,), jnp.int32),
            pltpu.VMEM((_NG, _G), jnp.int32),
            pltpu.VMEM((2, _D, _VC), jnp.float32),
            pltpu.VMEM((2 * _G // 4, 128), jnp.float32),
            pltpu.SemaphoreType.DMA,
            pltpu.SemaphoreType.DMA,
        ],
    )
    def k(emb_hbm, cat_hbm, tail_hbm, srt_hbm, bidx_hbm, idx_v, hist, cur,
          goff, sortv, bidx, chunks, stag, csem, fsem):
        wid = lax.axis_index("s") * 2 + lax.axis_index("c")
        f = wid

        @pl.when(f < _NF)
        def _work():
            pltpu.sync_copy(cat_hbm.at[f], idx_v)
            lanes = lax.iota(jnp.int32, 16)
            lane_off = lanes * _NB
            zeros16 = jnp.zeros((16,), jnp.int32)
            ones16 = jnp.ones((16,), jnp.int32)

            def zh(i, _):
                hist[pl.ds(i * 16, 16)] = zeros16
                return ()
            lax.fori_loop(0, 16 * _NB // 16, zh, (), unroll=False)

            def buckets_of(i):
                v = idx_v[i >> 3, pl.ds((i & 7) * 16, 16)]
                c = jnp.minimum(((v >> 8) * 683) >> 11, 130)
                c = jnp.where(v >= _TAILB, 131, c)
                base = jnp.where(c == 131, _TAILB, c * _VC)
                voff = v - base
                return c, voff

            def h1(i, _):
                c, _voff = buckets_of(i)
                slot = lane_off + c
                cnt = plsc.load_gather(hist, [slot])
                plsc.store_scatter(hist, [slot], cnt + ones16)
                return ()
            lax.fori_loop(0, _B // 16, h1, (), unroll=False)

            def red(g, carry):
                acc = zeros16

                def rl(l, a):
                    return a + hist[pl.ds(l * _NB + g * 16, 16)]
                acc = lax.fori_loop(0, 16, rl, acc, unroll=False)
                cs = plsc.cumsum(acc)
                excl = cs - acc + jnp.full((16,), carry, jnp.int32)
                goff[pl.ds(g * 16, 16)] = excl
                def wl(l, run):
                    cur[pl.ds(l * _NB + g * 16, 16)] = run
                    return run + hist[pl.ds(l * _NB + g * 16, 16)]
                lax.fori_loop(0, 16, wl, excl, unroll=False)
                return carry + cs[15]
            lax.fori_loop(0, _NB // 16, red, jnp.int32(0), unroll=False)

            bbase = f * _B

            def h2(i, _):
                c, voff = buckets_of(i)
                bglob = jnp.full((16,), bbase + i * 16, jnp.int32) + lanes
                packed = (bglob << 10) | voff
                slot = lane_off + c
                pos = plsc.load_gather(cur, [slot])
                plsc.store_scatter(sortv, [pos], packed)
                plsc.store_scatter(bidx, [pos >> 8, pos & (_G - 1)], bglob)
                plsc.store_scatter(cur, [slot], pos + 1)
                return ()
            lax.fori_loop(0, _B // 16, h2, (), unroll=False)

            pltpu.async_copy(emb_hbm.at[f, :, pl.ds(0, _VC)], chunks.at[0], csem)

            def start_chunk(c, buf):
                @pl.when(c <= 129)
                def _full():
                    pltpu.async_copy(
                        emb_hbm.at[f, :, pl.ds(pl.multiple_of(c * _VC, _VC), _VC)],
                        chunks.at[buf], csem)

                @pl.when(c == 130)
                def _short():
                    pltpu.async_copy(
                        emb_hbm.at[f, :, pl.ds(130 * _VC, _VLAST)],
                        chunks.at[buf, :, pl.ds(0, _VLAST)], csem)

                @pl.when(c == 131)
                def _tail():
                    pltpu.async_copy(
                        tail_hbm.at[f],
                        chunks.at[buf, :, pl.ds(0, 128)], csem)

            def wait_chunk(c, buf):
                @pl.when(c <= 129)
                def _full():
                    pltpu.make_async_copy(
                        emb_hbm.at[f, :, pl.ds(0, _VC)],
                        chunks.at[buf], csem).wait()

                @pl.when(c == 130)
                def _short():
                    pltpu.make_async_copy(
                        emb_hbm.at[f, :, pl.ds(130 * _VC, _VLAST)],
                        chunks.at[buf, :, pl.ds(0, _VLAST)], csem).wait()

                @pl.when(c == 131)
                def _tail():
                    pltpu.make_async_copy(
                        tail_hbm.at[f],
                        chunks.at[buf, :, pl.ds(0, 128)], csem).wait()

            def sweep(c, _):
                par = c & 1

                @pl.when(c + 1 < _NCHV)
                def _pref():
                    start_chunk(c + 1, 1 - par)
                wait_chunk(c, par)

                gv = goff[pl.ds(c, 16)]
                lo = gv[0]
                hi = gv[1]
                par16 = jnp.full((16,), par, jnp.int32)

                def item(kk, packed):
                    nxt = sortv[pl.ds(kk + 1, 16)][0]
                    voff16 = jnp.full((16,), packed & 1023, jnp.int32)
                    g0 = plsc.load_gather(chunks, [par16, lanes, voff16])
                    g1 = plsc.load_gather(
                        chunks, [par16, lanes + 16, voff16])
                    kw = kk & (2 * _G - 1)
                    srow = jnp.full((16,), kw >> 2, jnp.int32)
                    l0 = jnp.full((16,), (kw & 3) * _D, jnp.int32) + lanes
                    plsc.store_scatter(stag, [srow, l0], g0)
                    plsc.store_scatter(stag, [srow, l0 + 16], g1)

                    @pl.when((kk & (_G - 1)) == (_G - 1))
                    def _flush():
                        j = kk >> 8
                        jpar = j & 1
                        dsto = pl.multiple_of((f * _B + j * _G) // 4, 64)
                        pltpu.async_copy(
                            stag.at[pl.ds(jpar * (_G // 4), _G // 4)],
                            srt_hbm.at[pl.ds(dsto, _G // 4)],
                            fsem)

                        @pl.when(j >= 1)
                        def _drain():
                            pltpu.make_async_copy(
                                stag.at[pl.ds(0, _G // 4)],
                                srt_hbm.at[pl.ds(0, _G // 4)], fsem).wait()
                    return nxt
                lax.fori_loop(lo, hi, item, sortv[pl.ds(lo, 16)][0],
                              unroll=False)
                return ()
            lax.fori_loop(0, _NCHV, sweep, (), unroll=False)
            pltpu.make_async_copy(
                stag.at[pl.ds(0, _G // 4)],
                srt_hbm.at[pl.ds(0, _G // 4)], fsem).wait()
            pltpu.sync_copy(
                bidx, bidx_hbm.at[pl.ds(pl.multiple_of(f * _NG, _NG), _NG)])

    return k(embT3, catT, tail)


def _perm_scatter(srt, bidxh):
    mesh = plsc.VectorSubcoreMesh(core_axis_name="c", subcore_axis_name="s")
    _U = 128
    upw = _BNF // _U // 32

    @functools.partial(
        pl.kernel,
        mesh=mesh,
        out_type=jax.ShapeDtypeStruct((_BNF, _D), jnp.float32),
        compiler_params=pltpu.CompilerParams(
            use_tc_tiling_on_sc=False, needs_layout_passes=False),
        scratch_types=[
            pltpu.VMEM((upw, _U), jnp.int32),
            pltpu.VMEM((2, _U, _D), jnp.float32),
            pltpu.SemaphoreType.DMA,
            pltpu.SemaphoreType.DMA,
        ],
    )
    def k(srt_hbm, bidx_hbm, out_hbm, bidx_v, bufs, lsem, ssem):
        wid = lax.axis_index("s") * 2 + lax.axis_index("c")
        g0 = wid * upw
        pltpu.sync_copy(bidx_hbm.at[pl.ds(g0, upw)], bidx_v)
        pltpu.async_copy(srt_hbm.at[pl.ds(g0 * _U, _U)], bufs.at[0], lsem)

        def body(g, _):
            par = g & 1

            @pl.when(g >= 1)
            def _drain():
                pltpu.make_async_copy(
                    bufs.at[0], out_hbm.at[bidx_v.at[0]], ssem).wait()

            @pl.when(g + 1 < upw)
            def _pref():
                pltpu.async_copy(
                    srt_hbm.at[pl.ds((g0 + g + 1) * _U, _U)],
                    bufs.at[1 - par], lsem)
            pltpu.make_async_copy(
                srt_hbm.at[pl.ds(0, _U)], bufs.at[par], lsem).wait()
            pltpu.async_copy(bufs.at[par], out_hbm.at[bidx_v.at[g]], ssem)
            return ()
        lax.fori_loop(0, upw, body, (), unroll=False)
        pltpu.make_async_copy(
            bufs.at[0], out_hbm.at[bidx_v.at[0]], ssem).wait()

    return k(srt, bidxh)


def _tc_body(num_ref, gat_ref, wb0, bb0, wb1, bb1, wb2, bb2,
             w0a, w0i, bt0, wt1, bt1, wt2, bt2, wt3, bt3, wt4, bt4,
             out_ref):
    f32 = jnp.float32
    bf16 = jnp.bfloat16
    x = num_ref[...]
    x = jnp.maximum(jnp.dot(x, wb0[...], preferred_element_type=f32) + bb0[...], 0.0)
    x = jnp.maximum(jnp.dot(x.astype(bf16), wb1[...], preferred_element_type=f32) + bb1[...], 0.0)
    bot = jnp.maximum(jnp.dot(x.astype(bf16), wb2[...], preferred_element_type=f32) + bb2[...], 0.0)
    botb = bot.astype(bf16)
    gat = gat_ref[...].astype(bf16)
    A = jnp.concatenate([botb[None], gat], axis=0)
    inter = lax.dot_general(A, A, (((2,), (2,)), ((1,), (1,))),
                            preferred_element_type=f32)
    interf = inter.reshape(_R, _NI * _NI).astype(bf16)
    y = jnp.dot(botb, w0a[...], preferred_element_type=f32)
    y = y + jnp.dot(interf, w0i[...], preferred_element_type=f32)
    y = jnp.maximum(y + bt0[...], 0.0)
    y = jnp.maximum(jnp.dot(y.astype(bf16), wt1[...], preferred_element_type=f32) + bt1[...], 0.0)
    y = jnp.maximum(jnp.dot(y.astype(bf16), wt2[...], preferred_element_type=f32) + bt2[...], 0.0)
    y = jnp.maximum(jnp.dot(y.astype(bf16), wt3[...], preferred_element_type=f32) + bt3[...], 0.0)
    out_ref[...] = jnp.dot(y.astype(bf16), wt4[...], preferred_element_type=f32) + bt4[...]


def kernel(numerical_input, categorical_inputs, emb_tables,
           W_bot_0, b_bot_0, W_bot_1, b_bot_1, W_bot_2, b_bot_2,
           W_top_0, b_top_0, W_top_1, b_top_1, W_top_2, b_top_2,
           W_top_3, b_top_3, W_top_4, b_top_4):
    bf16 = jnp.bfloat16
    embT3 = jnp.transpose(emb_tables, (0, 2, 1))
    catT = jnp.transpose(categorical_inputs).reshape(_NF, 128, _B // 128)
    tail = jnp.pad(embT3[:, :, _TAILB:], ((0, 0), (0, 0), (0, 96)))
    srt4, bidxh = _sweep_gather(embT3, catT, tail)
    gathered = _perm_scatter(srt4.reshape(_BNF, _D),
                             bidxh.reshape(_BNF // 128, 128))
    gat3 = gathered.reshape(_NF, _B, _D)

    w0a = W_top_0[:_D].astype(bf16)
    w0i = jnp.zeros((_NI * _NI, W_top_0.shape[1]), bf16)
    w0i = w0i.at[_TRI].set(W_top_0[_D:_D + _TRI.shape[0]].astype(bf16))

    row = lambda b: b.reshape(1, -1)
    grid = _B // _R
    full = lambda a: pl.BlockSpec(a.shape, lambda i: (0,) * a.ndim)
    out = pl.pallas_call(
        _tc_body,
        grid=(grid,),
        in_specs=[
            pl.BlockSpec((_R, numerical_input.shape[1]), lambda i: (i, 0)),
            pl.BlockSpec((_NF, _R, _D), lambda i: (0, i, 0)),
            full(W_bot_0), full(row(b_bot_0)), full(W_bot_1), full(row(b_bot_1)),
            full(W_bot_2), full(row(b_bot_2)),
            full(w0a), full(w0i), full(row(b_top_0)),
            full(W_top_1), full(row(b_top_1)), full(W_top_2), full(row(b_top_2)),
            full(W_top_3), full(row(b_top_3)), full(W_top_4), full(row(b_top_4)),
        ],
        out_specs=pl.BlockSpec((_R, 1), lambda i: (i, 0)),
        out_shape=jax.ShapeDtypeStruct((_B, 1), jnp.float32),
    )(numerical_input.astype(bf16), gat3,
      W_bot_0.astype(bf16), row(b_bot_0), W_bot_1.astype(bf16), row(b_bot_1),
      W_bot_2.astype(bf16), row(b_bot_2),
      w0a, w0i, row(b_top_0), W_top_1.astype(bf16), row(b_top_1),
      W_top_2.astype(bf16), row(b_top_2),
      W_top_3.astype(bf16), row(b_top_3), W_top_4.astype(bf16), row(b_top_4))
    return out


# speedup vs baseline: 1.8183x; 1.0082x over previous
import functools

import numpy as np
import jax
import jax.numpy as jnp
from jax import lax
from jax.experimental import pallas as pl
from jax.experimental.pallas import tpu as pltpu
from jax.experimental.pallas import tpu_sc as plsc

_B = 16384
_NF = 26
_V = 100000
_D = 32
_NI = _NF + 1
_BNF = _B * _NF
_R = 1024

_VC = 768
_NCHV = 132
_VLAST = 128
_TAILB = 99968
_NB = 160
_G = 256
_NG = _B // _G

_TRI = np.array([i * _NI + j for i in range(_NI) for j in range(i)], dtype=np.int32)


def _sweep_gather(embT3, catT, tail):
    mesh = plsc.VectorSubcoreMesh(core_axis_name="c", subcore_axis_name="s")

    @functools.partial(
        pl.kernel,
        mesh=mesh,
        out_type=(jax.ShapeDtypeStruct((_BNF // 4, 128), jnp.float32),
                  jax.ShapeDtypeStruct((_NF * _NG, _G), jnp.int32)),
        compiler_params=pltpu.CompilerParams(needs_layout_passes=False),
        scratch_types=[
            pltpu.VMEM((128, 128), jnp.int32),
            pltpu.VMEM((16 * _NB,), jnp.int32),
            pltpu.VMEM((16 * _NB,), jnp.int32),
            pltpu.VMEM((_NB,), jnp.int32),
            pltpu.VMEM((_B + 16,), jnp.int32),
            pltpu.VMEM((_NG, _G), jnp.int32),
            pltpu.VMEM((2, _D, _VC), jnp.float32),
            pltpu.VMEM((2 * _G // 4, 128), jnp.float32),
            pltpu.SemaphoreType.DMA,
            pltpu.SemaphoreType.DMA,
        ],
    )
    def k(emb_hbm, cat_hbm, tail_hbm, srt_hbm, bidx_hbm, idx_v, hist, cur,
          goff, sortv, bidx, chunks, stag, csem, fsem):
        wid = lax.axis_index("s") * 2 + lax.axis_index("c")
        f = wid

        @pl.when(f < _NF)
        def _work():
            pltpu.sync_copy(cat_hbm.at[f], idx_v)
            lanes = lax.iota(jnp.int32, 16)
            lane_off = lanes * _NB
            zeros16 = jnp.zeros((16,), jnp.int32)
            ones16 = jnp.ones((16,), jnp.int32)

            def zh(i, _):
                hist[pl.ds(i * 16, 16)] = zeros16
                return ()
            lax.fori_loop(0, 16 * _NB // 16, zh, (), unroll=False)

            def buckets_of(i):
                v = idx_v[i >> 3, pl.ds((i & 7) * 16, 16)]
                c = jnp.minimum(((v >> 8) * 683) >> 11, 130)
                c = jnp.where(v >= _TAILB, 131, c)
                base = jnp.where(c == 131, _TAILB, c * _VC)
                voff = v - base
                return c, voff

            def h1(i, _):
                c, _voff = buckets_of(i)
                slot = lane_off + c
                cnt = plsc.load_gather(hist, [slot])
                plsc.store_scatter(hist, [slot], cnt + ones16)
                return ()
            lax.fori_loop(0, _B // 16, h1, (), unroll=False)

            def red(g, carry):
                acc = zeros16

                def rl(l, a):
                    return a + hist[pl.ds(l * _NB + g * 16, 16)]
                acc = lax.fori_loop(0, 16, rl, acc, unroll=False)
                cs = plsc.cumsum(acc)
                excl = cs - acc + jnp.full((16,), carry, jnp.int32)
                goff[pl.ds(g * 16, 16)] = excl
                def wl(l, run):
                    cur[pl.ds(l * _NB + g * 16, 16)] = run
                    return run + hist[pl.ds(l * _NB + g * 16, 16)]
                lax.fori_loop(0, 16, wl, excl, unroll=False)
                return carry + cs[15]
            lax.fori_loop(0, _NB // 16, red, jnp.int32(0), unroll=False)

            bbase = f * _B

            def h2(i, _):
                c, voff = buckets_of(i)
                bglob = jnp.full((16,), bbase + i * 16, jnp.int32) + lanes
                packed = (bglob << 10) | voff
                slot = lane_off + c
                pos = plsc.load_gather(cur, [slot])
                plsc.store_scatter(sortv, [pos], packed)
                plsc.store_scatter(bidx, [pos >> 8, pos & (_G - 1)], bglob)
                plsc.store_scatter(cur, [slot], pos + 1)
                return ()
            lax.fori_loop(0, _B // 16, h2, (), unroll=False)

            pltpu.async_copy(emb_hbm.at[f, :, pl.ds(0, _VC)], chunks.at[0], csem)

            def start_chunk(c, buf):
                @pl.when(c <= 129)
                def _full():
                    pltpu.async_copy(
                        emb_hbm.at[f, :, pl.ds(pl.multiple_of(c * _VC, _VC), _VC)],
                        chunks.at[buf], csem)

                @pl.when(c == 130)
                def _short():
                    pltpu.async_copy(
                        emb_hbm.at[f, :, pl.ds(130 * _VC, _VLAST)],
                        chunks.at[buf, :, pl.ds(0, _VLAST)], csem)

                @pl.when(c == 131)
                def _tail():
                    pltpu.async_copy(
                        tail_hbm.at[f],
                        chunks.at[buf, :, pl.ds(0, 128)], csem)

            def wait_chunk(c, buf):
                @pl.when(c <= 129)
                def _full():
                    pltpu.make_async_copy(
                        emb_hbm.at[f, :, pl.ds(0, _VC)],
                        chunks.at[buf], csem).wait()

                @pl.when(c == 130)
                def _short():
                    pltpu.make_async_copy(
                        emb_hbm.at[f, :, pl.ds(130 * _VC, _VLAST)],
                        chunks.at[buf, :, pl.ds(0, _VLAST)], csem).wait()

                @pl.when(c == 131)
                def _tail():
                    pltpu.make_async_copy(
                        tail_hbm.at[f],
                        chunks.at[buf, :, pl.ds(0, 128)], csem).wait()

            def sweep(c, _):
                par = c & 1

                @pl.when(c + 1 < _NCHV)
                def _pref():
                    start_chunk(c + 1, 1 - par)
                wait_chunk(c, par)

                gv = goff[pl.ds(c, 16)]
                lo = gv[0]
                hi = gv[1]
                par16 = jnp.full((16,), par, jnp.int32)

                def item(kk, packed):
                    nxt = sortv[pl.ds(kk + 1, 16)][0]
                    voff16 = jnp.full((16,), packed & 1023, jnp.int32)
                    g0 = plsc.load_gather(chunks, [par16, lanes, voff16])
                    g1 = plsc.load_gather(
                        chunks, [par16, lanes + 16, voff16])
                    kw = kk & (2 * _G - 1)
                    srow = jnp.full((16,), kw >> 2, jnp.int32)
                    l0 = jnp.full((16,), (kw & 3) * _D, jnp.int32) + lanes
                    plsc.store_scatter(stag, [srow, l0], g0)
                    plsc.store_scatter(stag, [srow, l0 + 16], g1)

                    @pl.when((kk & (_G - 1)) == (_G - 1))
                    def _flush():
                        j = kk >> 8
                        jpar = j & 1
                        dsto = pl.multiple_of((f * _B + j * _G) // 4, 64)
                        pltpu.async_copy(
                            stag.at[pl.ds(jpar * (_G // 4), _G // 4)],
                            srt_hbm.at[pl.ds(dsto, _G // 4)],
                            fsem)

                        @pl.when(j >= 1)
                        def _drain():
                            pltpu.make_async_copy(
                                stag.at[pl.ds(0, _G // 4)],
                                srt_hbm.at[pl.ds(0, _G // 4)], fsem).wait()
                    return nxt
                lax.fori_loop(lo, hi, item, sortv[pl.ds(lo, 16)][0],
                              unroll=False)
                return ()
            lax.fori_loop(0, _NCHV, sweep, (), unroll=False)
            pltpu.make_async_copy(
                stag.at[pl.ds(0, _G // 4)],
                srt_hbm.at[pl.ds(0, _G // 4)], fsem).wait()
            pltpu.sync_copy(
                bidx, bidx_hbm.at[pl.ds(pl.multiple_of(f * _NG, _NG), _NG)])

    return k(embT3, catT, tail)


def _perm_scatter(srt, bidxh):
    mesh = plsc.VectorSubcoreMesh(core_axis_name="c", subcore_axis_name="s")
    _U = 128
    upw = _BNF // _U // 32

    @functools.partial(
        pl.kernel,
        mesh=mesh,
        out_type=jax.ShapeDtypeStruct((_BNF, _D), jnp.float32),
        compiler_params=pltpu.CompilerParams(
            use_tc_tiling_on_sc=False, needs_layout_passes=False),
        scratch_types=[
            pltpu.VMEM((upw, _U), jnp.int32),
            pltpu.VMEM((2, _U, _D), jnp.float32),
            pltpu.SemaphoreType.DMA,
            pltpu.SemaphoreType.DMA,
        ],
    )
    def k(srt_hbm, bidx_hbm, out_hbm, bidx_v, bufs, lsem, ssem):
        wid = lax.axis_index("s") * 2 + lax.axis_index("c")
        g0 = wid * upw
        pltpu.sync_copy(bidx_hbm.at[pl.ds(g0, upw)], bidx_v)
        pltpu.async_copy(srt_hbm.at[pl.ds(g0 * _U, _U)], bufs.at[0], lsem)

        def body(g, _):
            par = g & 1

            @pl.when(g >= 1)
            def _drain():
                pltpu.make_async_copy(
                    bufs.at[0], out_hbm.at[bidx_v.at[0]], ssem).wait()

            @pl.when(g + 1 < upw)
            def _pref():
                pltpu.async_copy(
                    srt_hbm.at[pl.ds((g0 + g + 1) * _U, _U)],
                    bufs.at[1 - par], lsem)
            pltpu.make_async_copy(
                srt_hbm.at[pl.ds(0, _U)], bufs.at[par], lsem).wait()
            pltpu.async_copy(bufs.at[par], out_hbm.at[bidx_v.at[g]], ssem)
            return ()
        lax.fori_loop(0, upw, body, (), unroll=False)
        pltpu.make_async_copy(
            bufs.at[0], out_hbm.at[bidx_v.at[0]], ssem).wait()

    return k(srt, bidxh)


def _tc_body(num_ref, gat_ref, wb0, bb0, wb1, bb1, wb2, bb2,
             w0a, w0i, bt0, wt1, bt1, wt2, bt2, wt3, bt3, wt4, bt4,
             out_ref):
    f32 = jnp.float32
    bf16 = jnp.bfloat16
    x = num_ref[...]
    x = jnp.maximum(jnp.dot(x, wb0[...], preferred_element_type=f32) + bb0[...], 0.0)
    x = jnp.maximum(jnp.dot(x.astype(bf16), wb1[...], preferred_element_type=f32) + bb1[...], 0.0)
    bot = jnp.maximum(jnp.dot(x.astype(bf16), wb2[...], preferred_element_type=f32) + bb2[...], 0.0)
    botb = bot.astype(bf16)
    gat = gat_ref[...].astype(bf16)
    A = jnp.concatenate([botb[None], gat], axis=0)
    inter = lax.dot_general(A, A, (((2,), (2,)), ((1,), (1,))),
                            preferred_element_type=f32)
    interf = inter.reshape(_R, _NI * _NI).astype(bf16)
    y = jnp.dot(botb, w0a[...], preferred_element_type=f32)
    y = y + jnp.dot(interf, w0i[...], preferred_element_type=f32)
    y = jnp.maximum(y + bt0[...], 0.0)
    y = jnp.maximum(jnp.dot(y.astype(bf16), wt1[...], preferred_element_type=f32) + bt1[...], 0.0)
    y = jnp.maximum(jnp.dot(y.astype(bf16), wt2[...], preferred_element_type=f32) + bt2[...], 0.0)
    y = jnp.maximum(jnp.dot(y.astype(bf16), wt3[...], preferred_element_type=f32) + bt3[...], 0.0)
    out_ref[...] = jnp.dot(y.astype(bf16), wt4[...], preferred_element_type=f32) + bt4[...]


def kernel(numerical_input, categorical_inputs, emb_tables,
           W_bot_0, b_bot_0, W_bot_1, b_bot_1, W_bot_2, b_bot_2,
           W_top_0, b_top_0, W_top_1, b_top_1, W_top_2, b_top_2,
           W_top_3, b_top_3, W_top_4, b_top_4):
    bf16 = jnp.bfloat16
    embT3 = jnp.transpose(emb_tables, (0, 2, 1))
    catT = jnp.transpose(categorical_inputs).reshape(_NF, 128, _B // 128)
    tail = jnp.pad(embT3[:, :, _TAILB:], ((0, 0), (0, 0), (0, 96)))
    srt4, bidxh = _sweep_gather(embT3, catT, tail)
    gathered = _perm_scatter(srt4.reshape(_BNF, _D),
                             bidxh.reshape(_BNF // 128, 128))
    gat3 = gathered.reshape(_NF, _B, _D)

    w0a = W_top_0[:_D].astype(bf16)
    w0i = jnp.zeros((_NI * _NI, W_top_0.shape[1]), bf16)
    w0i = w0i.at[_TRI].set(W_top_0[_D:_D + _TRI.shape[0]].astype(bf16))

    row = lambda b: b.reshape(1, -1)
    grid = _B // _R
    full = lambda a: pl.BlockSpec(a.shape, lambda i: (0,) * a.ndim)
    out = pl.pallas_call(
        _tc_body,
        grid=(grid,),
        in_specs=[
            pl.BlockSpec((_R, numerical_input.shape[1]), lambda i: (i, 0)),
            pl.BlockSpec((_NF, _R, _D), lambda i: (0, i, 0)),
            full(W_bot_0), full(row(b_bot_0)), full(W_bot_1), full(row(b_bot_1)),
            full(W_bot_2), full(row(b_bot_2)),
            full(w0a), full(w0i), full(row(b_top_0)),
            full(W_top_1), full(row(b_top_1)), full(W_top_2), full(row(b_top_2)),
            full(W_top_3), full(row(b_top_3)), full(W_top_4), full(row(b_top_4)),
        ],
        out_specs=pl.BlockSpec((_R, 1), lambda i: (i, 0)),
        out_shape=jax.ShapeDtypeStruct((_B, 1), jnp.float32),
    )(numerical_input.astype(bf16), gat3,
      W_bot_0.astype(bf16), row(b_bot_0), W_bot_1.astype(bf16), row(b_bot_1),
      W_bot_2.astype(bf16), row(b_bot_2),
      w0a, w0i, row(b_top_0), W_top_1.astype(bf16), row(b_top_1),
      W_top_2.astype(bf16), row(b_top_2),
      W_top_3.astype(bf16), row(b_top_3), W_top_4.astype(bf16), row(b_top_4))
    return out


# speedup vs baseline: 2.7592x; 1.5175x over previous
import functools

import numpy as np
import jax
import jax.numpy as jnp
from jax import lax
from jax.experimental import pallas as pl
from jax.experimental.pallas import tpu as pltpu
from jax.experimental.pallas import tpu_sc as plsc

_B = 16384
_NF = 26
_V = 100000
_D = 32
_NI = _NF + 1
_BNF = _B * _NF
_R = 1024

_VC = 768
_NCHV = 132
_VLAST = 128
_TAILB = 99968
_NB = 160
_G = 256
_NG = _B // _G

_TRI = np.array([i * _NI + j for i in range(_NI) for j in range(i)], dtype=np.int32)


def _sweep_gather(embT3, catT, tail):
    mesh = plsc.VectorSubcoreMesh(core_axis_name="c", subcore_axis_name="s")

    @functools.partial(
        pl.kernel,
        mesh=mesh,
        out_type=(jax.ShapeDtypeStruct((_BNF // 4, 128), jnp.float32),
                  jax.ShapeDtypeStruct((_NF * _NG, _G), jnp.int32)),
        compiler_params=pltpu.CompilerParams(needs_layout_passes=False),
        scratch_types=[
            pltpu.VMEM((128, 128), jnp.int32),
            pltpu.VMEM((16 * _NB,), jnp.int32),
            pltpu.VMEM((16 * _NB,), jnp.int32),
            pltpu.VMEM((_NB,), jnp.int32),
            pltpu.VMEM((_B + 16,), jnp.int32),
            pltpu.VMEM((_NG, _G), jnp.int32),
            pltpu.VMEM((2, _D, _VC), jnp.float32),
            pltpu.VMEM((2 * _G // 4, 128), jnp.float32),
            pltpu.SemaphoreType.DMA,
            pltpu.SemaphoreType.DMA,
        ],
    )
    def k(emb_hbm, cat_hbm, tail_hbm, srt_hbm, bidx_hbm, idx_v, hist, cur,
          goff, sortv, bidx, chunks, stag, csem, fsem):
        wid = lax.axis_index("s") * 2 + lax.axis_index("c")
        f = wid

        @pl.when(f < _NF)
        def _work():
            pltpu.sync_copy(cat_hbm.at[f], idx_v)
            lanes = lax.iota(jnp.int32, 16)
            lane_off = lanes * _NB
            zeros16 = jnp.zeros((16,), jnp.int32)
            ones16 = jnp.ones((16,), jnp.int32)

            def zh(i, _):
                hist[pl.ds(i * 16, 16)] = zeros16
                return ()
            lax.fori_loop(0, 16 * _NB // 16, zh, (), unroll=False)

            def buckets_of(i):
                v = idx_v[i >> 3, pl.ds((i & 7) * 16, 16)]
                c = jnp.minimum(((v >> 8) * 683) >> 11, 130)
                c = jnp.where(v >= _TAILB, 131, c)
                base = jnp.where(c == 131, _TAILB, c * _VC)
                voff = v - base
                return c, voff

            def h1(i, _):
                c, _voff = buckets_of(i)
                slot = lane_off + c
                cnt = plsc.load_gather(hist, [slot])
                plsc.store_scatter(hist, [slot], cnt + ones16)
                return ()
            lax.fori_loop(0, _B // 16, h1, (), unroll=False)

            def red(g, carry):
                acc = zeros16

                def rl(l, a):
                    return a + hist[pl.ds(l * _NB + g * 16, 16)]
                acc = lax.fori_loop(0, 16, rl, acc, unroll=False)
                cs = plsc.cumsum(acc)
                excl = cs - acc + jnp.full((16,), carry, jnp.int32)
                goff[pl.ds(g * 16, 16)] = excl
                def wl(l, run):
                    cur[pl.ds(l * _NB + g * 16, 16)] = run
                    return run + hist[pl.ds(l * _NB + g * 16, 16)]
                lax.fori_loop(0, 16, wl, excl, unroll=False)
                return carry + cs[15]
            lax.fori_loop(0, _NB // 16, red, jnp.int32(0), unroll=False)

            bbase = f * _B

            def h2(i, _):
                c, voff = buckets_of(i)
                bglob = jnp.full((16,), bbase + i * 16, jnp.int32) + lanes
                packed = (bglob << 10) | voff
                slot = lane_off + c
                pos = plsc.load_gather(cur, [slot])
                plsc.store_scatter(sortv, [pos], packed)
                plsc.store_scatter(bidx, [pos >> 8, pos & (_G - 1)], bglob)
                plsc.store_scatter(cur, [slot], pos + 1)
                return ()
            lax.fori_loop(0, _B // 16, h2, (), unroll=False)

            pltpu.async_copy(emb_hbm.at[f, :, pl.ds(0, _VC)], chunks.at[0], csem)

            def start_chunk(c, buf):
                @pl.when(c <= 129)
                def _full():
                    pltpu.async_copy(
                        emb_hbm.at[f, :, pl.ds(pl.multiple_of(c * _VC, _VC), _VC)],
                        chunks.at[buf], csem)

                @pl.when(c == 130)
                def _short():
                    pltpu.async_copy(
                        emb_hbm.at[f, :, pl.ds(130 * _VC, _VLAST)],
                        chunks.at[buf, :, pl.ds(0, _VLAST)], csem)

                @pl.when(c == 131)
                def _tail():
                    pltpu.async_copy(
                        tail_hbm.at[f],
                        chunks.at[buf, :, pl.ds(0, 128)], csem)

            def wait_chunk(c, buf):
                @pl.when(c <= 129)
                def _full():
                    pltpu.make_async_copy(
                        emb_hbm.at[f, :, pl.ds(0, _VC)],
                        chunks.at[buf], csem).wait()

                @pl.when(c == 130)
                def _short():
                    pltpu.make_async_copy(
                        emb_hbm.at[f, :, pl.ds(130 * _VC, _VLAST)],
                        chunks.at[buf, :, pl.ds(0, _VLAST)], csem).wait()

                @pl.when(c == 131)
                def _tail():
                    pltpu.make_async_copy(
                        tail_hbm.at[f],
                        chunks.at[buf, :, pl.ds(0, 128)], csem).wait()

            def sweep(c, _):
                par = c & 1

                @pl.when(c + 1 < _NCHV)
                def _pref():
                    start_chunk(c + 1, 1 - par)
                wait_chunk(c, par)

                gv = goff[pl.ds(c, 16)]
                lo = gv[0]
                hi = gv[1]
                par16 = jnp.full((16,), par, jnp.int32)

                def seg(kcur):
                    kend = jnp.minimum(hi, (kcur & ~(_G - 1)) + _G)

                    @functools.partial(
                        plsc.parallel_loop, kcur, kend, unroll=4)
                    def _items(kk):
                        packed = sortv[pl.ds(kk, 16)][0]
                        voff16 = jnp.full((16,), packed & 1023, jnp.int32)
                        g0 = plsc.load_gather(chunks, [par16, lanes, voff16])
                        g1 = plsc.load_gather(
                            chunks, [par16, lanes + 16, voff16])
                        kw = kk & (2 * _G - 1)
                        srow = jnp.full((16,), kw >> 2, jnp.int32)
                        l0 = jnp.full((16,), (kw & 3) * _D, jnp.int32) + lanes
                        plsc.store_scatter(stag, [srow, l0], g0)
                        plsc.store_scatter(stag, [srow, l0 + 16], g1)

                    @pl.when((kend & (_G - 1)) == 0)
                    def _flush():
                        j = (kend >> 8) - 1
                        jpar = j & 1
                        dsto = pl.multiple_of((f * _B + j * _G) // 4, 64)
                        pltpu.async_copy(
                            stag.at[pl.ds(jpar * (_G // 4), _G // 4)],
                            srt_hbm.at[pl.ds(dsto, _G // 4)],
                            fsem)

                        @pl.when(j >= 1)
                        def _drain():
                            pltpu.make_async_copy(
                                stag.at[pl.ds(0, _G // 4)],
                                srt_hbm.at[pl.ds(0, _G // 4)], fsem).wait()
                    return kend
                lax.while_loop(lambda k: k < hi, seg, lo)
                return ()
            lax.fori_loop(0, _NCHV, sweep, (), unroll=False)
            pltpu.make_async_copy(
                stag.at[pl.ds(0, _G // 4)],
                srt_hbm.at[pl.ds(0, _G // 4)], fsem).wait()
            pltpu.sync_copy(
                bidx, bidx_hbm.at[pl.ds(pl.multiple_of(f * _NG, _NG), _NG)])

    return k(embT3, catT, tail)


def _perm_scatter(srt, bidxh):
    mesh = plsc.VectorSubcoreMesh(core_axis_name="c", subcore_axis_name="s")
    _U = 128
    upw = _BNF // _U // 32

    @functools.partial(
        pl.kernel,
        mesh=mesh,
        out_type=jax.ShapeDtypeStruct((_BNF, _D), jnp.float32),
        compiler_params=pltpu.CompilerParams(
            use_tc_tiling_on_sc=False, needs_layout_passes=False),
        scratch_types=[
            pltpu.VMEM((upw, _U), jnp.int32),
            pltpu.VMEM((2, _U, _D), jnp.float32),
            pltpu.SemaphoreType.DMA,
            pltpu.SemaphoreType.DMA,
        ],
    )
    def k(srt_hbm, bidx_hbm, out_hbm, bidx_v, bufs, lsem, ssem):
        wid = lax.axis_index("s") * 2 + lax.axis_index("c")
        g0 = wid * upw
        pltpu.sync_copy(bidx_hbm.at[pl.ds(g0, upw)], bidx_v)
        pltpu.async_copy(srt_hbm.at[pl.ds(g0 * _U, _U)], bufs.at[0], lsem)

        def body(g, _):
            par = g & 1

            @pl.when(g >= 1)
            def _drain():
                pltpu.make_async_copy(
                    bufs.at[0], out_hbm.at[bidx_v.at[0]], ssem).wait()

            @pl.when(g + 1 < upw)
            def _pref():
                pltpu.async_copy(
                    srt_hbm.at[pl.ds((g0 + g + 1) * _U, _U)],
                    bufs.at[1 - par], lsem)
            pltpu.make_async_copy(
                srt_hbm.at[pl.ds(0, _U)], bufs.at[par], lsem).wait()
            pltpu.async_copy(bufs.at[par], out_hbm.at[bidx_v.at[g]], ssem)
            return ()
        lax.fori_loop(0, upw, body, (), unroll=False)
        pltpu.make_async_copy(
            bufs.at[0], out_hbm.at[bidx_v.at[0]], ssem).wait()

    return k(srt, bidxh)


def _tc_body(num_ref, gat_ref, wb0, bb0, wb1, bb1, wb2, bb2,
             w0a, w0i, bt0, wt1, bt1, wt2, bt2, wt3, bt3, wt4, bt4,
             out_ref):
    f32 = jnp.float32
    bf16 = jnp.bfloat16
    x = num_ref[...]
    x = jnp.maximum(jnp.dot(x, wb0[...], preferred_element_type=f32) + bb0[...], 0.0)
    x = jnp.maximum(jnp.dot(x.astype(bf16), wb1[...], preferred_element_type=f32) + bb1[...], 0.0)
    bot = jnp.maximum(jnp.dot(x.astype(bf16), wb2[...], preferred_element_type=f32) + bb2[...], 0.0)
    botb = bot.astype(bf16)
    gat = gat_ref[...].astype(bf16)
    A = jnp.concatenate([botb[None], gat], axis=0)
    inter = lax.dot_general(A, A, (((2,), (2,)), ((1,), (1,))),
                            preferred_element_type=f32)
    interf = inter.reshape(_R, _NI * _NI).astype(bf16)
    y = jnp.dot(botb, w0a[...], preferred_element_type=f32)
    y = y + jnp.dot(interf, w0i[...], preferred_element_type=f32)
    y = jnp.maximum(y + bt0[...], 0.0)
    y = jnp.maximum(jnp.dot(y.astype(bf16), wt1[...], preferred_element_type=f32) + bt1[...], 0.0)
    y = jnp.maximum(jnp.dot(y.astype(bf16), wt2[...], preferred_element_type=f32) + bt2[...], 0.0)
    y = jnp.maximum(jnp.dot(y.astype(bf16), wt3[...], preferred_element_type=f32) + bt3[...], 0.0)
    out_ref[...] = jnp.dot(y.astype(bf16), wt4[...], preferred_element_type=f32) + bt4[...]


def kernel(numerical_input, categorical_inputs, emb_tables,
           W_bot_0, b_bot_0, W_bot_1, b_bot_1, W_bot_2, b_bot_2,
           W_top_0, b_top_0, W_top_1, b_top_1, W_top_2, b_top_2,
           W_top_3, b_top_3, W_top_4, b_top_4):
    bf16 = jnp.bfloat16
    embT3 = jnp.transpose(emb_tables, (0, 2, 1))
    catT = jnp.transpose(categorical_inputs).reshape(_NF, 128, _B // 128)
    tail = jnp.pad(embT3[:, :, _TAILB:], ((0, 0), (0, 0), (0, 96)))
    srt4, bidxh = _sweep_gather(embT3, catT, tail)
    gathered = _perm_scatter(srt4.reshape(_BNF, _D),
                             bidxh.reshape(_BNF // 128, 128))
    gat3 = gathered.reshape(_NF, _B, _D)

    w0a = W_top_0[:_D].astype(bf16)
    w0i = jnp.zeros((_NI * _NI, W_top_0.shape[1]), bf16)
    w0i = w0i.at[_TRI].set(W_top_0[_D:_D + _TRI.shape[0]].astype(bf16))

    row = lambda b: b.reshape(1, -1)
    grid = _B // _R
    full = lambda a: pl.BlockSpec(a.shape, lambda i: (0,) * a.ndim)
    out = pl.pallas_call(
        _tc_body,
        grid=(grid,),
        in_specs=[
            pl.BlockSpec((_R, numerical_input.shape[1]), lambda i: (i, 0)),
            pl.BlockSpec((_NF, _R, _D), lambda i: (0, i, 0)),
            full(W_bot_0), full(row(b_bot_0)), full(W_bot_1), full(row(b_bot_1)),
            full(W_bot_2), full(row(b_bot_2)),
            full(w0a), full(w0i), full(row(b_top_0)),
            full(W_top_1), full(row(b_top_1)), full(W_top_2), full(row(b_top_2)),
            full(W_top_3), full(row(b_top_3)), full(W_top_4), full(row(b_top_4)),
        ],
        out_specs=pl.BlockSpec((_R, 1), lambda i: (i, 0)),
        out_shape=jax.ShapeDtypeStruct((_B, 1), jnp.float32),
    )(numerical_input.astype(bf16), gat3,
      W_bot_0.astype(bf16), row(b_bot_0), W_bot_1.astype(bf16), row(b_bot_1),
      W_bot_2.astype(bf16), row(b_bot_2),
      w0a, w0i, row(b_top_0), W_top_1.astype(bf16), row(b_top_1),
      W_top_2.astype(bf16), row(b_top_2),
      W_top_3.astype(bf16), row(b_top_3), W_top_4.astype(bf16), row(b_top_4))
    return out
